# Initial kernel scaffold; baseline (speedup 1.0000x reference)
import jax
import jax.numpy as jnp
from jax.experimental import pallas as pl


def kernel(tokens, index_weight, w1, b1, token_values):
    raise NotImplementedError("write your pallas kernel here")


# speedup vs baseline: 3.9340x; 3.9340x over previous
import jax
import jax.numpy as jnp
from jax import lax
from jax.experimental import pallas as pl
from jax.experimental.pallas import tpu as pltpu
from jax.experimental.pallas import tpu_sc as plsc

DIM = 64
NC = 2
NS = 16
NW = NC * NS
LANES = 16

IDXW = 128
SLICES = 8
CHUNK = IDXW * SLICES


def _body(tok_hbm, iw_hbm, tv_hbm, w_hbm, b_hbm, out_hbm,
          idx_v, vals_v, rows_v, w_v, b_v, sem):
    wid = lax.axis_index("s") * NC + lax.axis_index("c")
    nrows_w = tok_hbm.shape[0] // NW
    nchunks = nrows_w // SLICES
    row0 = wid * nrows_w

    pltpu.sync_copy(w_hbm, w_v)
    pltpu.sync_copy(b_hbm, b_v)
    wv = [w_v[pl.ds(g * LANES, LANES)] for g in range(DIM // LANES)]
    bv = [b_v[pl.ds(g * LANES, LANES)] for g in range(DIM // LANES)]

    def chunk_body(c, carry):
        pltpu.sync_copy(tok_hbm.at[pl.ds(row0 + c * SLICES, SLICES)], idx_v)
        cps = []
        for j in range(SLICES):
            cps.append(pltpu.async_copy(
                iw_hbm.at[idx_v.at[j]],
                rows_v.at[pl.ds(j * IDXW, IDXW)], sem))
            cps.append(pltpu.async_copy(
                tv_hbm.at[idx_v.at[j]],
                vals_v.at[pl.ds(j * IDXW, IDXW)], sem))
        for cp in cps:
            cp.wait()

        def blk_body(i, rcarry):
            vvec = vals_v[pl.ds(i * LANES, LANES)]
            for k in range(LANES):
                r = i * LANES + k
                val = vvec[k]
                for g in range(DIM // LANES):
                    sl = pl.ds(g * LANES, LANES)
                    rows_v[r, sl] = rows_v[r, sl] + (val * wv[g] + bv[g])
            return rcarry
        lax.fori_loop(0, CHUNK // LANES, blk_body, 0)

        pltpu.sync_copy(
            rows_v, out_hbm.at[pl.ds((row0 + c * SLICES) * IDXW, CHUNK)])
        return carry

    lax.fori_loop(0, nchunks, chunk_body, 0)


def kernel(tokens, index_weight, w1, b1, token_values):
    bsz, fields = tokens.shape
    n = bsz * fields
    tok = tokens.reshape(n // IDXW, IDXW)
    wvec = w1[:, 0]

    run = pl.kernel(
        _body,
        out_type=jax.ShapeDtypeStruct((n, DIM), jnp.float32),
        mesh=plsc.VectorSubcoreMesh(core_axis_name="c", subcore_axis_name="s"),
        scratch_types=[
            pltpu.VMEM((SLICES, IDXW), jnp.int32),
            pltpu.VMEM((CHUNK,), jnp.float32),
            pltpu.VMEM((CHUNK, DIM), jnp.float32),
            pltpu.VMEM((DIM,), jnp.float32),
            pltpu.VMEM((DIM,), jnp.float32),
            pltpu.SemaphoreType.DMA,
        ],
        compiler_params=pltpu.CompilerParams(use_tc_tiling_on_sc=False),
    )
    out = run(tok, index_weight, token_values, wvec, b1)
    return out.reshape(bsz, fields, DIM)


# speedup vs baseline: 3.9586x; 1.0063x over previous
import jax
import jax.numpy as jnp
from jax import lax
from jax.experimental import pallas as pl
from jax.experimental.pallas import tpu as pltpu
from jax.experimental.pallas import tpu_sc as plsc

DIM = 64
NC = 2
NS = 16
NW = NC * NS
LANES = 16

CB = 64


def _body(tok_hbm, iw_hbm, tv_hbm, w_hbm, b_hbm, out_hbm,
          idx_v, vals_v, rows_v, w_v, b_v, sem):
    bsz, fields = tok_hbm.shape
    wid = lax.axis_index("s") * NC + lax.axis_index("c")
    b_per_w = bsz // NW
    nchunks = b_per_w // CB
    b_base = wid * b_per_w

    pltpu.sync_copy(w_hbm, w_v)
    pltpu.sync_copy(b_hbm, b_v)
    wv = [w_v[pl.ds(g * LANES, LANES)] for g in range(DIM // LANES)]
    bv = [b_v[pl.ds(g * LANES, LANES)] for g in range(DIM // LANES)]

    def chunk_body(c, carry):
        b0 = b_base + c * CB
        pltpu.sync_copy(tok_hbm.at[pl.ds(b0, CB)], idx_v)
        cps = []
        for j in range(CB):
            cps.append(pltpu.async_copy(
                iw_hbm.at[idx_v.at[j]], rows_v.at[j], sem))
            cps.append(pltpu.async_copy(
                tv_hbm.at[idx_v.at[j]], vals_v.at[j], sem))
        for cp in cps:
            cp.wait()

        def blk_body(i, rcarry):
            vlo = vals_v[i, pl.ds(0, LANES)]
            vhi = vals_v[i, pl.ds(fields - LANES, LANES)]
            for f in range(fields):
                if f < LANES:
                    val = vlo[f]
                else:
                    val = vhi[f - (fields - LANES)]
                for g in range(DIM // LANES):
                    gsl = pl.ds(g * LANES, LANES)
                    rows_v[i, f, gsl] = rows_v[i, f, gsl] + (val * wv[g] + bv[g])
            return rcarry
        lax.fori_loop(0, CB, blk_body, 0)

        pltpu.sync_copy(rows_v, out_hbm.at[pl.ds(b0, CB)])
        return carry

    lax.fori_loop(0, nchunks, chunk_body, 0)


def kernel(tokens, index_weight, w1, b1, token_values):
    bsz, fields = tokens.shape

    run = pl.kernel(
        _body,
        out_type=jax.ShapeDtypeStruct((bsz, fields, DIM), jnp.float32),
        mesh=plsc.VectorSubcoreMesh(core_axis_name="c", subcore_axis_name="s"),
        scratch_types=[
            pltpu.VMEM((CB, fields), jnp.int32),
            pltpu.VMEM((CB, fields), jnp.float32),
            pltpu.VMEM((CB, fields, DIM), jnp.float32),
            pltpu.VMEM((DIM,), jnp.float32),
            pltpu.VMEM((DIM,), jnp.float32),
            pltpu.SemaphoreType.DMA,
        ],
        compiler_params=pltpu.CompilerParams(use_tc_tiling_on_sc=False),
    )
    return run(tokens, index_weight, token_values, w1[:, 0], b1)
